# Initial kernel scaffold; baseline (speedup 1.0000x reference)
import jax
import jax.numpy as jnp
from jax.experimental import pallas as pl


def kernel(rays_o, rays_d, xyz, features_dc, features_rest, opacity):
    raise NotImplementedError("write your pallas kernel here")


# speedup vs baseline: 1.9088x; 1.9088x over previous
import functools

import jax
import jax.numpy as jnp
from jax.experimental import pallas as pl
from jax.experimental.pallas import tpu as pltpu
from jax.experimental.pallas import tpu_sc as plsc

NPTS = 100000
NRAYS = 1024
KSEL = 10
SLOTS = 16
TILE = 2048
NTILES = (NPTS + TILE - 1) // TILE
NPAD = NTILES * TILE

NCORES = 2
NSUB = 16
NW = NCORES * NSUB
BPW = NRAYS // NW
IDX_CHUNK = 128
NCHUNK = (BPW * SLOTS) // IDX_CHUNK


def _lex_less(ad, ai, bd, bi):
    return (ad < bd) | ((ad == bd) & (ai < bi))


def _tc_body(ro_ref, rd_ref, xyzt_ref, wb_ref, idx_ref, sd_ref, si_ref):
    t = pl.program_id(0)

    @pl.when(t == 0)
    def _init():
        sd_ref[...] = jnp.full((NRAYS, SLOTS), jnp.inf, jnp.float32)
        si_ref[...] = jnp.zeros((NRAYS, SLOTS), jnp.int32)

    c = ro_ref[...] + 3.0 * rd_ref[...]
    cn = jnp.sum(c * c, axis=1, keepdims=True)
    xt = xyzt_ref[...]
    pn = jnp.sum(xt * xt, axis=0, keepdims=True)
    g = jax.lax.dot_general(
        c.astype(jnp.bfloat16), xt.astype(jnp.bfloat16),
        (((1,), (0,)), ((), ())),
        preferred_element_type=jnp.float32,
    )
    d2 = cn + pn - 2.0 * g
    dist = jnp.sqrt(jnp.maximum(d2, 1e-12))

    lane_t = jax.lax.broadcasted_iota(jnp.int32, (NRAYS, TILE), 1)
    li16 = jax.lax.broadcasted_iota(jnp.int32, (NRAYS, SLOTS), 1)
    base = t * TILE

    tv = jnp.full((NRAYS, SLOTS), jnp.inf, jnp.float32)
    ti = jnp.zeros((NRAYS, SLOTS), jnp.int32)
    for k in range(KSEL):
        m = jnp.min(dist, axis=1, keepdims=True)
        am = jnp.min(jnp.where(dist == m, lane_t, TILE), axis=1,
                     keepdims=True)
        dist = jnp.where(lane_t == am, jnp.inf, dist)
        slot = SLOTS - 1 - k
        tv = jnp.where(li16 == slot, m, tv)
        ti = jnp.where(li16 == slot, am + base, ti)

    rd_ = sd_ref[...]
    ri_ = si_ref[...]
    take = _lex_less(tv, ti, rd_, ri_)
    md = jnp.where(take, tv, rd_)
    mi = jnp.where(take, ti, ri_)

    for d in (8, 4, 2, 1):
        lo = (li16 & d) == 0
        pd = jnp.where(lo, jnp.roll(md, -d, axis=1), jnp.roll(md, d, axis=1))
        pi = jnp.where(lo, jnp.roll(mi, -d, axis=1), jnp.roll(mi, d, axis=1))
        less = _lex_less(pd, pi, md, mi)
        takep = jnp.logical_xor(less, ~lo)
        md = jnp.where(takep, pd, md)
        mi = jnp.where(takep, pi, mi)

    sd_ref[...] = md
    si_ref[...] = mi

    wb_ref[...] = jnp.where(li16 < KSEL, jnp.exp(md * -0.1), 0.0)
    idx_ref[...] = mi


@jax.jit
def _tc_topk(ro8, rd8, xyzt):
    return pl.pallas_call(
        _tc_body,
        grid=(NTILES,),
        in_specs=[
            pl.BlockSpec((NRAYS, 8), lambda i: (0, 0)),
            pl.BlockSpec((NRAYS, 8), lambda i: (0, 0)),
            pl.BlockSpec((8, TILE), lambda i: (0, i)),
        ],
        out_specs=[
            pl.BlockSpec((NRAYS, SLOTS), lambda i: (0, 0)),
            pl.BlockSpec((NRAYS, SLOTS), lambda i: (0, 0)),
        ],
        out_shape=[
            jax.ShapeDtypeStruct((NRAYS, SLOTS), jnp.float32),
            jax.ShapeDtypeStruct((NRAYS, SLOTS), jnp.int32),
        ],
        scratch_shapes=[
            pltpu.VMEM((NRAYS, SLOTS), jnp.float32),
            pltpu.VMEM((NRAYS, SLOTS), jnp.int32),
        ],
        compiler_params=pltpu.CompilerParams(
            dimension_semantics=("arbitrary",),
        ),
    )(ro8, rd8, xyzt)


NCHUNK_USED = 3
GLEN = NCHUNK_USED * IDX_CHUNK


def _sc_body(tc0_hbm, tc1_hbm, tc2_hbm, top_hbm, wbt_hbm, idx_hbm, out_hbm,
             idx_v, wb_v, c0_v, c1_v, c2_v, op_v, out_v, sem):
    wid = jax.lax.axis_index("s") * NCORES + jax.lax.axis_index("c")
    base = wid * BPW

    pltpu.sync_copy(idx_hbm.at[wid], idx_v)
    for k in range(KSEL):
        pltpu.sync_copy(wbt_hbm.at[pl.ds(k * NRAYS + base, BPW)],
                        wb_v.at[pl.ds(k * BPW, BPW)])
    for j in range(NCHUNK_USED):
        for tab, buf in ((tc0_hbm, c0_v), (tc1_hbm, c1_v),
                         (tc2_hbm, c2_v), (top_hbm, op_v)):
            pltpu.async_copy(
                tab.at[idx_v.at[j]],
                buf.at[pl.ds(j * IDX_CHUNK, IDX_CHUNK)],
                sem,
            ).wait()

    for grp in range(BPW // 16):
        j0 = grp * 16
        acc0 = jnp.zeros((16,), jnp.float32)
        acc1 = jnp.zeros((16,), jnp.float32)
        acc2 = jnp.zeros((16,), jnp.float32)
        wsum = jnp.zeros((16,), jnp.float32)
        for k in range(KSEL):
            p = k * BPW + j0
            wb = wb_v[pl.ds(p, 16)]
            c0 = c0_v[pl.ds(p, 16)]
            c1 = c1_v[pl.ds(p, 16)]
            c2 = c2_v[pl.ds(p, 16)]
            op = op_v[pl.ds(p, 16)]
            w = wb * (1.0 / (1.0 + jnp.exp(-op)))
            acc0 = acc0 + w * (1.0 / (1.0 + jnp.exp(-c0)))
            acc1 = acc1 + w * (1.0 / (1.0 + jnp.exp(-c1)))
            acc2 = acc2 + w * (1.0 / (1.0 + jnp.exp(-c2)))
            wsum = wsum + w
        den = wsum + 1e-8
        out_v[pl.ds(0 * BPW + j0, 16)] = acc0 / den
        out_v[pl.ds(1 * BPW + j0, 16)] = acc1 / den
        out_v[pl.ds(2 * BPW + j0, 16)] = acc2 / den

    for ch in range(3):
        pltpu.sync_copy(out_v.at[pl.ds(ch * BPW, BPW)],
                        out_hbm.at[pl.ds(ch * NRAYS + base, BPW)])


@jax.jit
def _sc_combine(tc0, tc1, tc2, top, wbt, idx3):
    mesh = plsc.VectorSubcoreMesh(
        core_axis_name="c", subcore_axis_name="s",
        num_cores=NCORES, num_subcores=NSUB,
    )
    return pl.kernel(
        _sc_body,
        out_type=jax.ShapeDtypeStruct((3 * NRAYS,), jnp.float32),
        mesh=mesh,
        scratch_types=[
            pltpu.VMEM((NCHUNK, IDX_CHUNK), jnp.int32),
            pltpu.VMEM((KSEL * BPW,), jnp.float32),
            pltpu.VMEM((GLEN,), jnp.float32),
            pltpu.VMEM((GLEN,), jnp.float32),
            pltpu.VMEM((GLEN,), jnp.float32),
            pltpu.VMEM((GLEN,), jnp.float32),
            pltpu.VMEM((3 * BPW,), jnp.float32),
            pltpu.SemaphoreType.DMA,
        ],
    )(tc0, tc1, tc2, top, wbt, idx3)


def kernel(rays_o, rays_d, xyz, features_dc, features_rest, opacity):
    ro8 = jnp.zeros((NRAYS, 8), jnp.float32).at[:, :3].set(rays_o)
    rd8 = jnp.zeros((NRAYS, 8), jnp.float32).at[:, :3].set(rays_d)
    xyzt = jnp.full((8, NPAD), 0.0, jnp.float32)
    xyzt = xyzt.at[:3, :NPTS].set(xyz.T)
    xyzt = xyzt.at[:3, NPTS:].set(1e15)

    wb, idx = _tc_topk(ro8, rd8, xyzt)

    tc0 = features_dc[:, 0, 0]
    tc1 = features_dc[:, 0, 1]
    tc2 = features_dc[:, 0, 2]
    top = opacity[:, 0]
    wbt = wb.T.reshape(-1)
    idx3 = jnp.transpose(idx.reshape(NW, BPW, SLOTS), (0, 2, 1)).reshape(
        NW, NCHUNK, IDX_CHUNK)

    outp = _sc_combine(tc0, tc1, tc2, top, wbt, idx3)
    return outp.reshape(3, NRAYS).T
